# Initial kernel scaffold; baseline (speedup 1.0000x reference)
import jax
import jax.numpy as jnp
from jax.experimental import pallas as pl


def kernel(inputs, label, weight):
    raise NotImplementedError("write your pallas kernel here")


# speedup vs baseline: 5.1538x; 5.1538x over previous
import jax
import jax.numpy as jnp
from jax import lax
from jax.experimental import pallas as pl
from jax.experimental.pallas import tpu as pltpu

_N_EMB = 1024
_DIM = 64
_B = 16384
_BLK = 1024
_GRID = _B // _BLK
_COMMIT = 0.25
_DIVERGE = 0.1


def _vq_body(x_ref, lab_ref, w_ref, loss_ref, quant_ref, perp_ref, enc_ref,
             acc_ref, cnt_ref):
    i = pl.program_id(0)

    @pl.when(i == 0)
    def _init():
        acc_ref[0] = 0.0
        acc_ref[1] = 0.0
        cnt_ref[...] = jnp.zeros_like(cnt_ref)

    x = x_ref[...]
    w = w_ref[...]
    lab = lab_ref[...]

    s = lax.dot_general(x, w, (((1,), (1,)), ((), ())))
    ones_d = jnp.ones((1, _DIM), jnp.float32)
    w2row = lax.dot_general(ones_d, w * w, (((1,), (1,)), ((), ())))
    nox = w2row - 2.0 * s

    col = lax.broadcasted_iota(jnp.int32, (_BLK, _N_EMB), 1)
    enc = (col == lab).astype(jnp.float32)
    enc_ref[...] = enc

    x2 = jnp.sum(x * x, axis=1, keepdims=True)
    nox_lab = jnp.sum(nox * enc, axis=1, keepdims=True)
    dmin = jnp.min(nox, axis=1, keepdims=True)
    amin = jnp.min(jnp.where(nox == dmin, col, _N_EMB), axis=1, keepdims=True)
    ind = (amin != lab).astype(jnp.float32)

    q_par = jnp.sum(x2 + nox_lab)
    x_par = jnp.sum(ind * (x2 + dmin))

    quant_ref[...] = lax.dot_general(enc, w, (((1,), (0,)), ((), ())))

    acc_ref[0] += q_par
    acc_ref[1] += x_par
    ones_b = jnp.ones((1, _BLK), jnp.float32)
    cnt_ref[...] += lax.dot_general(ones_b, enc, (((1,), (0,)), ((), ())))

    @pl.when(i == _GRID - 1)
    def _fini():
        denom = float(_B * _DIM)
        loss = ((1.0 + _COMMIT) * acc_ref[0] - (1.0 + _DIVERGE) * acc_ref[1]) / denom
        loss_ref[...] = jnp.full((8, 128), loss, jnp.float32)
        probs = cnt_ref[...] / float(_B)
        ent = -jnp.sum(probs * jnp.log(probs + 1e-10))
        perp_ref[...] = jnp.full((8, 128), jnp.exp(ent), jnp.float32)


def kernel(inputs, label, weight):
    lab2d = label.reshape(_B, 1).astype(jnp.int32)

    loss_a, quant, perp_a, enc = pl.pallas_call(
        _vq_body,
        grid=(_GRID,),
        in_specs=[
            pl.BlockSpec((_BLK, _DIM), lambda i: (i, 0)),
            pl.BlockSpec((_BLK, 1), lambda i: (i, 0)),
            pl.BlockSpec((_N_EMB, _DIM), lambda i: (0, 0)),
        ],
        out_specs=[
            pl.BlockSpec((8, 128), lambda i: (0, 0)),
            pl.BlockSpec((_BLK, _DIM), lambda i: (i, 0)),
            pl.BlockSpec((8, 128), lambda i: (0, 0)),
            pl.BlockSpec((_BLK, _N_EMB), lambda i: (i, 0)),
        ],
        out_shape=[
            jax.ShapeDtypeStruct((8, 128), jnp.float32),
            jax.ShapeDtypeStruct((_B, _DIM), jnp.float32),
            jax.ShapeDtypeStruct((8, 128), jnp.float32),
            jax.ShapeDtypeStruct((_B, _N_EMB), jnp.float32),
        ],
        scratch_shapes=[
            pltpu.SMEM((2,), jnp.float32),
            pltpu.VMEM((1, _N_EMB), jnp.float32),
        ],
        compiler_params=pltpu.CompilerParams(
            dimension_semantics=("arbitrary",),
        ),
    )(inputs, lab2d, weight)

    return loss_a[0, 0], quant, perp_a[0, 0], enc


# speedup vs baseline: 6.3958x; 1.2410x over previous
import jax
import jax.numpy as jnp
from jax import lax
from jax.experimental import pallas as pl
from jax.experimental.pallas import tpu as pltpu

_N_EMB = 1024
_DIM = 64
_B = 16384
_BLK = 1024
_GRID = _B // _BLK
_COMMIT = 0.25
_DIVERGE = 0.1


def _vq_body(x_ref, lab_ref, w_ref, loss_ref, quant_ref, perp_ref, enc_ref,
             acc_ref, cnt_ref, xa_ref, wa_ref):
    i = pl.program_id(0)

    @pl.when(i == 0)
    def _init():
        acc_ref[0] = 0.0
        acc_ref[1] = 0.0
        cnt_ref[...] = jnp.zeros_like(cnt_ref)
        w = w_ref[...]
        w2col = jnp.sum(w * w, axis=1, keepdims=True)
        lane_w = lax.broadcasted_iota(jnp.int32, (_N_EMB, _DIM), 1)
        wa_ref[:, 0:_DIM] = -2.0 * w
        wa_ref[:, _DIM:2 * _DIM] = jnp.where(lane_w == 0, w2col, 0.0)
        lane_x = lax.broadcasted_iota(jnp.int32, (_BLK, _DIM), 1)
        xa_ref[:, _DIM:2 * _DIM] = jnp.where(lane_x == 0, 1.0, 0.0)

    x = x_ref[...]
    lab = lab_ref[...]
    xa_ref[:, 0:_DIM] = x

    nox = lax.dot_general(xa_ref[...], wa_ref[...], (((1,), (1,)), ((), ())))

    col = lax.broadcasted_iota(jnp.int32, (_BLK, _N_EMB), 1)
    enc = (col == lab).astype(jnp.float32)
    enc_ref[...] = enc

    quant = lax.dot_general(enc, w_ref[...], (((1,), (0,)), ((), ())))
    quant_ref[...] = quant
    d = quant - x
    q_par = jnp.sum(d * d)

    x2 = jnp.sum(x * x, axis=1, keepdims=True)
    dmin = jnp.min(nox, axis=1, keepdims=True)
    amin = jnp.min(jnp.where(nox == dmin, col, _N_EMB), axis=1, keepdims=True)
    ind = (amin != lab).astype(jnp.float32)
    x_par = jnp.sum(ind * (x2 + dmin))

    acc_ref[0] += q_par
    acc_ref[1] += x_par
    ones_b = jnp.ones((1, _BLK), jnp.float32)
    cnt_ref[...] += lax.dot_general(ones_b, enc, (((1,), (0,)), ((), ())))

    @pl.when(i == _GRID - 1)
    def _fini():
        denom = float(_B * _DIM)
        loss = ((1.0 + _COMMIT) * acc_ref[0] - (1.0 + _DIVERGE) * acc_ref[1]) / denom
        loss_ref[...] = jnp.full((8, 128), loss, jnp.float32)
        probs = cnt_ref[...] / float(_B)
        ent = -jnp.sum(probs * jnp.log(probs + 1e-10))
        perp_ref[...] = jnp.full((8, 128), jnp.exp(ent), jnp.float32)


def kernel(inputs, label, weight):
    lab2d = label.reshape(_B, 1).astype(jnp.int32)

    loss_a, quant, perp_a, enc = pl.pallas_call(
        _vq_body,
        grid=(_GRID,),
        in_specs=[
            pl.BlockSpec((_BLK, _DIM), lambda i: (i, 0)),
            pl.BlockSpec((_BLK, 1), lambda i: (i, 0)),
            pl.BlockSpec((_N_EMB, _DIM), lambda i: (0, 0)),
        ],
        out_specs=[
            pl.BlockSpec((8, 128), lambda i: (0, 0)),
            pl.BlockSpec((_BLK, _DIM), lambda i: (i, 0)),
            pl.BlockSpec((8, 128), lambda i: (0, 0)),
            pl.BlockSpec((_BLK, _N_EMB), lambda i: (i, 0)),
        ],
        out_shape=[
            jax.ShapeDtypeStruct((8, 128), jnp.float32),
            jax.ShapeDtypeStruct((_B, _DIM), jnp.float32),
            jax.ShapeDtypeStruct((8, 128), jnp.float32),
            jax.ShapeDtypeStruct((_B, _N_EMB), jnp.float32),
        ],
        scratch_shapes=[
            pltpu.SMEM((2,), jnp.float32),
            pltpu.VMEM((1, _N_EMB), jnp.float32),
            pltpu.VMEM((_BLK, 2 * _DIM), jnp.float32),
            pltpu.VMEM((_N_EMB, 2 * _DIM), jnp.float32),
        ],
        compiler_params=pltpu.CompilerParams(
            dimension_semantics=("arbitrary",),
        ),
    )(inputs, lab2d, weight)

    return loss_a[0, 0], quant, perp_a[0, 0], enc
